# baseline (speedup 1.0000x reference)
import functools

import jax
import jax.numpy as jnp
from jax import lax
from jax.experimental import pallas as pl
from jax.experimental.pallas import tpu as pltpu
from jax.experimental.pallas import tpu_sc as plsc

D_SIZES = (128, 256, 128, 512)
D_OFFS = (0, 128, 384, 512)
D_SUM = 1024
N_OUT = 1024
B_TOK = 16 * 2048

NUM_CORES = 2
NUM_SUBCORES = 16
NUM_WORKERS = NUM_CORES * NUM_SUBCORES

N_SLICES = 2
H_TOK = B_TOK // N_SLICES
TOK_PER_W = H_TOK // NUM_WORKERS
CHUNKS = tuple(min(c, TOK_PER_W) for c in (128, 128, 128, 64))

BLK = 1024


def _sc_gather(xT, t0, t1, t2, t3, sbase):
    mesh = plsc.VectorSubcoreMesh(core_axis_name="c", subcore_axis_name="s")
    out_type = tuple(
        jax.ShapeDtypeStruct((H_TOK, d), jnp.float32) for d in D_SIZES
    )
    scratch = (
        [pltpu.VMEM((TOK_PER_W,), jnp.int32)]
        + [pltpu.VMEM((c, d), jnp.float32) for c, d in zip(CHUNKS, D_SIZES)]
        + [pltpu.SemaphoreType.DMA]
    )

    @functools.partial(
        pl.kernel, mesh=mesh, out_type=out_type, scratch_types=scratch
    )
    def k(x_hbm, t0_hbm, t1_hbm, t2_hbm, t3_hbm, g0, g1, g2, g3,
          idx_v, r0, r1, r2, r3, sem):
        wid = lax.axis_index("s") * NUM_CORES + lax.axis_index("c")
        base = wid * TOK_PER_W
        tabs = (t0_hbm, t1_hbm, t2_hbm, t3_hbm)
        gouts = (g0, g1, g2, g3)
        rbufs = (r0, r1, r2, r3)
        for i in range(4):
            pltpu.sync_copy(
                x_hbm.at[i, pl.ds(sbase + base, TOK_PER_W)], idx_v
            )
            chunk = CHUNKS[i]

            def body(c, carry, i=i, chunk=chunk):
                start = c * chunk
                pltpu.async_copy(
                    tabs[i].at[idx_v.at[pl.ds(start, chunk)]], rbufs[i], sem
                ).wait()
                pltpu.sync_copy(
                    rbufs[i], gouts[i].at[pl.ds(base + start, chunk)]
                )
                return carry

            lax.fori_loop(0, TOK_PER_W // chunk, body, 0)

    return k(xT, t0, t1, t2, t3)


def _tc_matmul_slice(acc, gs, W, b2, s):
    blk0 = s * H_TOK // BLK
    grid = (H_TOK // BLK,)
    in_specs = [
        pl.BlockSpec(memory_space=pl.ANY),
    ] + [
        pl.BlockSpec((BLK, d), lambda i: (i, 0)) for d in D_SIZES
    ] + [
        pl.BlockSpec((D_SUM, N_OUT), lambda i: (0, 0)),
        pl.BlockSpec((1, N_OUT), lambda i: (0, 0)),
    ]
    out_specs = pl.BlockSpec((BLK, N_OUT), lambda i: (i + blk0, 0))

    def body(a, g0, g1, g2, g3, w, bb, o):
        del a
        blocks = (g0, g1, g2, g3)
        acc_ = bb[...].astype(jnp.float32)
        for i in range(4):
            acc_ = acc_ + jnp.dot(
                blocks[i][...].astype(jnp.bfloat16),
                w[D_OFFS[i]:D_OFFS[i] + D_SIZES[i], :],
                preferred_element_type=jnp.float32,
            )
        o[...] = acc_

    return pl.pallas_call(
        body,
        grid=grid,
        in_specs=in_specs,
        out_specs=out_specs,
        out_shape=jax.ShapeDtypeStruct((B_TOK, N_OUT), jnp.float32),
        input_output_aliases={0: 0},
    )(acc, *gs, W, b2)


def _tc_matmul_first(gs, W, b2):
    grid = (H_TOK // BLK,)
    in_specs = [
        pl.BlockSpec((BLK, d), lambda i: (i, 0)) for d in D_SIZES
    ] + [
        pl.BlockSpec((D_SUM, N_OUT), lambda i: (0, 0)),
        pl.BlockSpec((1, N_OUT), lambda i: (0, 0)),
    ]
    out_specs = pl.BlockSpec((BLK, N_OUT), lambda i: (i, 0))

    def body(g0, g1, g2, g3, w, bb, o):
        blocks = (g0, g1, g2, g3)
        acc_ = bb[...].astype(jnp.float32)
        for i in range(4):
            acc_ = acc_ + jnp.dot(
                blocks[i][...].astype(jnp.bfloat16),
                w[D_OFFS[i]:D_OFFS[i] + D_SIZES[i], :],
                preferred_element_type=jnp.float32,
            )
        o[...] = acc_

    return pl.pallas_call(
        body,
        grid=grid,
        in_specs=in_specs,
        out_specs=out_specs,
        out_shape=jax.ShapeDtypeStruct((B_TOK, N_OUT), jnp.float32),
    )(*gs, W, b2)


def kernel(x, t0, t1, t2, t3, W, b):
    bsz, seq, _ = x.shape
    xT = jnp.transpose(x.reshape(-1, 4).astype(jnp.int32))
    W16 = W.astype(jnp.bfloat16)
    b2 = b.reshape(1, N_OUT)
    gs_slices = [
        _sc_gather(xT, t0, t1, t2, t3, s * H_TOK) for s in range(N_SLICES)
    ]
    out = _tc_matmul_first(gs_slices[0], W16, b2)
    for s in range(1, N_SLICES):
        out = _tc_matmul_slice(out, gs_slices[s], W16, b2, s)
    return out.reshape(bsz, seq, N_OUT)


# speedup vs baseline: 1.0813x; 1.0813x over previous
import functools

import jax
import jax.numpy as jnp
from jax import lax
from jax.experimental import pallas as pl
from jax.experimental.pallas import tpu as pltpu
from jax.experimental.pallas import tpu_sc as plsc

D_SIZES = (128, 256, 128, 512)
D_OFFS = (0, 128, 384, 512)
D_SUM = 1024
N_OUT = 1024
B_TOK = 16 * 2048

NUM_CORES = 2
NUM_SUBCORES = 16
NUM_WORKERS = NUM_CORES * NUM_SUBCORES

N_SLICES = 4
H_TOK = B_TOK // N_SLICES
TOK_PER_W = H_TOK // NUM_WORKERS
CHUNKS = tuple(min(c, TOK_PER_W) for c in (128, 64, 128, 64))

BLK = 1024


def _sc_gather(xT, t0, t1, t2, t3, sbase):
    mesh = plsc.VectorSubcoreMesh(core_axis_name="c", subcore_axis_name="s")
    out_type = tuple(
        jax.ShapeDtypeStruct((H_TOK, d), jnp.float32) for d in D_SIZES
    )
    scratch = (
        [pltpu.VMEM((TOK_PER_W,), jnp.int32) for _ in range(4)]
        + [pltpu.VMEM((CHUNKS[i], D_SIZES[i]), jnp.float32) for i in range(3)]
        + [pltpu.VMEM((CHUNKS[3], D_SIZES[3]), jnp.float32) for _ in range(2)]
        + [pltpu.SemaphoreType.DMA, pltpu.SemaphoreType.DMA]
    )

    @functools.partial(
        pl.kernel, mesh=mesh, out_type=out_type, scratch_types=scratch
    )
    def k(x_hbm, t0_hbm, t1_hbm, t2_hbm, t3_hbm, g0, g1, g2, g3,
          i0, i1, i2, i3, r0, r1, r2, r3a, r3b, sem_g, sem_w):
        wid = lax.axis_index("s") * NUM_CORES + lax.axis_index("c")
        base = wid * TOK_PER_W
        tabs = (t0_hbm, t1_hbm, t2_hbm, t3_hbm)
        gouts = (g0, g1, g2, g3)
        idxs = (i0, i1, i2, i3)
        for i in range(4):
            pltpu.sync_copy(
                x_hbm.at[i, pl.ds(sbase + base, TOK_PER_W)], idxs[i]
            )

        def bufof(i, c):
            if i == 3:
                return (r3a, r3b)[c % 2]
            return (r0, r1, r2)[i]

        jobs = []
        for i in range(4):
            nch = TOK_PER_W // CHUNKS[i]
            for c in range(nch):
                jobs.append(((c + 0.5) / nch, i, c))
        jobs.sort()

        fifo = []
        prev = None
        for _, i, c in jobs:
            buf = bufof(i, c)
            if any(b is buf for _, b in fifo):
                while True:
                    h, b = fifo.pop(0)
                    h.wait()
                    if b is buf:
                        break
            gh = pltpu.async_copy(
                tabs[i].at[idxs[i].at[pl.ds(c * CHUNKS[i], CHUNKS[i])]],
                buf, sem_g,
            )
            if prev is not None:
                pgh, pi, pc, pbuf = prev
                pgh.wait()
                wh = pltpu.async_copy(
                    pbuf,
                    gouts[pi].at[pl.ds(base + pc * CHUNKS[pi], CHUNKS[pi])],
                    sem_w,
                )
                fifo.append((wh, pbuf))
            prev = (gh, i, c, buf)
        pgh, pi, pc, pbuf = prev
        pgh.wait()
        wh = pltpu.async_copy(
            pbuf, gouts[pi].at[pl.ds(base + pc * CHUNKS[pi], CHUNKS[pi])],
            sem_w,
        )
        fifo.append((wh, pbuf))
        for h, _ in fifo:
            h.wait()

    return k(xT, t0, t1, t2, t3)


def _tc_matmul_slice(acc, gs, W, b2, s):
    blk0 = s * H_TOK // BLK
    grid = (H_TOK // BLK,)
    in_specs = [
        pl.BlockSpec(memory_space=pl.ANY),
    ] + [
        pl.BlockSpec((BLK, d), lambda i: (i, 0)) for d in D_SIZES
    ] + [
        pl.BlockSpec((D_SUM, N_OUT), lambda i: (0, 0)),
        pl.BlockSpec((1, N_OUT), lambda i: (0, 0)),
    ]
    out_specs = pl.BlockSpec((BLK, N_OUT), lambda i: (i + blk0, 0))

    def body(a, g0, g1, g2, g3, w, bb, o):
        del a
        blocks = (g0, g1, g2, g3)
        acc_ = bb[...].astype(jnp.float32)
        for i in range(4):
            acc_ = acc_ + jnp.dot(
                blocks[i][...].astype(jnp.bfloat16),
                w[D_OFFS[i]:D_OFFS[i] + D_SIZES[i], :],
                preferred_element_type=jnp.float32,
            )
        o[...] = acc_

    return pl.pallas_call(
        body,
        grid=grid,
        in_specs=in_specs,
        out_specs=out_specs,
        out_shape=jax.ShapeDtypeStruct((B_TOK, N_OUT), jnp.float32),
        input_output_aliases={0: 0},
    )(acc, *gs, W, b2)


def _tc_matmul_first(gs, W, b2):
    grid = (H_TOK // BLK,)
    in_specs = [
        pl.BlockSpec((BLK, d), lambda i: (i, 0)) for d in D_SIZES
    ] + [
        pl.BlockSpec((D_SUM, N_OUT), lambda i: (0, 0)),
        pl.BlockSpec((1, N_OUT), lambda i: (0, 0)),
    ]
    out_specs = pl.BlockSpec((BLK, N_OUT), lambda i: (i, 0))

    def body(g0, g1, g2, g3, w, bb, o):
        blocks = (g0, g1, g2, g3)
        acc_ = bb[...].astype(jnp.float32)
        for i in range(4):
            acc_ = acc_ + jnp.dot(
                blocks[i][...].astype(jnp.bfloat16),
                w[D_OFFS[i]:D_OFFS[i] + D_SIZES[i], :],
                preferred_element_type=jnp.float32,
            )
        o[...] = acc_

    return pl.pallas_call(
        body,
        grid=grid,
        in_specs=in_specs,
        out_specs=out_specs,
        out_shape=jax.ShapeDtypeStruct((B_TOK, N_OUT), jnp.float32),
    )(*gs, W, b2)


def kernel(x, t0, t1, t2, t3, W, b):
    bsz, seq, _ = x.shape
    xT = jnp.transpose(x.reshape(-1, 4).astype(jnp.int32))
    W16 = W.astype(jnp.bfloat16)
    b2 = b.reshape(1, N_OUT)
    gs_slices = [
        _sc_gather(xT, t0, t1, t2, t3, s * H_TOK) for s in range(N_SLICES)
    ]
    out = _tc_matmul_first(gs_slices[0], W16, b2)
    for s in range(1, N_SLICES):
        out = _tc_matmul_slice(out, gs_slices[s], W16, b2, s)
    return out.reshape(bsz, seq, N_OUT)
